# baseline (speedup 1.0000x reference)
import functools

import jax
import jax.numpy as jnp
from jax import lax
from jax.experimental import pallas as pl
from jax.experimental.pallas import tpu as pltpu
from jax.experimental.pallas import tpu_sc as plsc

D_MODEL = 1024
SCALE = 32.0
NC, NS, L = 2, 16, 16
NW = NC * NS
B = 4 * 8192
BPW = B // NW
CH = 16
NCHUNK = BPW // CH
RING = 6
PD = 2
PEEL = RING - PD
VPR = D_MODEL // L

assert (NCHUNK - PEEL) % RING == 0

_mesh = plsc.VectorSubcoreMesh(core_axis_name="c", subcore_axis_name="s")


@functools.partial(
    pl.kernel,
    out_type=jax.ShapeDtypeStruct((B, D_MODEL), jnp.float32),
    mesh=_mesh,
    scratch_types=[
        pltpu.VMEM((BPW,), jnp.int32),
    ] + [pltpu.VMEM((CH, D_MODEL), jnp.float32)] * RING
      + [pltpu.SemaphoreType.DMA] * (2 * RING),
)
def _embed_sc(x_hbm, table_hbm, out_hbm, idx_v, *rest):
    bufs = rest[:RING]
    gsems = rest[RING:2 * RING]
    ssems = rest[2 * RING:]

    wid = lax.axis_index("s") * NC + lax.axis_index("c")
    base = wid * BPW
    pltpu.sync_copy(x_hbm.at[pl.ds(base, BPW)], idx_v)

    def issue_gather(c, b):
        off = pl.multiple_of(c * CH, 8)
        pltpu.async_copy(table_hbm.at[idx_v.at[pl.ds(off, CH)]], bufs[b], gsems[b])

    def wait_gather(b):
        pltpu.make_async_copy(table_hbm.at[pl.ds(0, CH)], bufs[b], gsems[b]).wait()

    def scale_buf(b):
        buf = bufs[b]

        @plsc.parallel_loop(0, CH)
        def _(r):
            for j in range(VPR):
                buf[r, pl.ds(j * L, L)] = buf[r, pl.ds(j * L, L)] * SCALE

    def issue_scatter(c, b):
        off = pl.multiple_of(c * CH, 8)
        pltpu.async_copy(bufs[b], out_hbm.at[pl.ds(base + off, CH)], ssems[b])

    def wait_scatter(b):
        pltpu.make_async_copy(bufs[b], out_hbm.at[pl.ds(0, CH)], ssems[b]).wait()

    def visit(c, b, wait_wr, prefetch):
        nb = (b + PD) % RING
        if wait_wr:
            wait_scatter(nb)
        if prefetch:
            issue_gather(c + PD, nb)

        wait_gather(b)
        scale_buf(b)
        issue_scatter(c, b)

    for c in range(PD):
        issue_gather(c, c)

    for c in range(PEEL):
        visit(c, c, wait_wr=False, prefetch=True)

    def outer(t, carry):
        for i in range(RING):
            c = PEEL + t * RING + i
            b = (PEEL + i) % RING

            @pl.when(c + PD < NCHUNK)
            def _():
                nb = (b + PD) % RING
                wait_scatter(nb)
                issue_gather(c + PD, nb)

            wait_gather(b)
            scale_buf(b)
            issue_scatter(c, b)

        return carry

    lax.fori_loop(0, (NCHUNK - PEEL) // RING, outer, 0)

    for c in range(NCHUNK - RING, NCHUNK):
        wait_scatter(c % RING)


def kernel(x, embedding):
    xf = x.reshape(-1).astype(jnp.int32)
    out = _embed_sc(xf, embedding)
    return out.reshape(x.shape[0], x.shape[1], D_MODEL)
